# baseline (speedup 1.0000x reference)
import functools

import jax
import jax.numpy as jnp
from jax import lax
from jax.experimental import pallas as pl
from jax.experimental.pallas import tpu as pltpu
from jax.experimental.pallas import tpu_sc as plsc

N_NODES = 10000
N_EDGES = 320000
D = 128
DEG_W = 128
EPS = 1e-5

NC = 2
NS = 16
NW = NC * NS
CHUNK = 128
N_CHUNKS = N_EDGES // CHUNK
BASE_CHUNKS = N_CHUNKS // NW
EXTRA = N_CHUNKS - BASE_CHUNKS * NW
EPW = N_EDGES // NW
FULL = EPW // CHUNK
TAIL = EPW - FULL * CHUNK
R_MAIN = 624
R_TAIL_BASE = NS * R_MAIN
R_TAIL = N_NODES - R_TAIL_BASE


def _worker_ids():
    c = lax.axis_index("c")
    s = lax.axis_index("s")
    return c, s, s * NC + c


def _zero_shared(shared, stage, r0):
    for k in range(4):
        pltpu.sync_copy(stage, shared.at[pl.ds(r0 + k * CHUNK, CHUNK)])
    pltpu.sync_copy(stage.at[pl.ds(0, R_MAIN - 4 * CHUNK)],
                    shared.at[pl.ds(r0 + 4 * CHUNK, R_MAIN - 4 * CHUNK)])


def _flush_shared(shared, stage, out_hbm, c, r0):
    def one(base, n):
        pltpu.sync_copy(shared.at[pl.ds(base, n)], stage.at[pl.ds(0, n)])
        pltpu.sync_copy(stage.at[pl.ds(0, n)], out_hbm.at[c, pl.ds(base, n)])

    for k in range(4):
        one(r0 + k * CHUNK, CHUNK)
    one(r0 + 4 * CHUNK, R_MAIN - 4 * CHUNK)
    return one


def _sc_segment_sum(x, src, dst, z128):
    mesh = plsc.VectorSubcoreMesh(core_axis_name="c", subcore_axis_name="s")

    @functools.partial(
        pl.kernel,
        out_type=jax.ShapeDtypeStruct((NC, N_NODES, D), jnp.float32),
        mesh=mesh,
        scratch_types=[
            pltpu.VMEM((CHUNK,), jnp.int32),
            pltpu.VMEM((CHUNK,), jnp.int32),
            pltpu.VMEM((CHUNK,), jnp.int32),
            pltpu.VMEM((CHUNK,), jnp.int32),
            pltpu.VMEM((CHUNK,), jnp.int32),
            pltpu.VMEM((CHUNK, D), jnp.float32),
            pltpu.VMEM((CHUNK, D), jnp.float32),
            pltpu.VMEM((TAIL,), jnp.int32),
            pltpu.VMEM((TAIL,), jnp.int32),
            pltpu.VMEM_SHARED((N_NODES, D), jnp.float32),
            pltpu.SemaphoreType.DMA,
            pltpu.SemaphoreType.DMA,
            pltpu.SemaphoreType.DMA,
            pltpu.SemaphoreType.DMA,
            pltpu.SemaphoreType.DMA,
            pltpu.SemaphoreType.DMA,
            pltpu.SemaphoreType.DMA,
            pltpu.SemaphoreType.DMA,
            pltpu.SemaphoreType.DMA,
            pltpu.SemaphoreType.DMA,
        ],
    )
    def seg_sum(x_hbm, src_hbm, dst_hbm, z128_hbm, acc_out,
                sv0, sv1, dv0, dv1, dv2, rw0, rw1, srct, dstt, accs,
                isr0, isr1, idt0, idt1, idt2, gs0, gs1, ss0, ss1, ss2):
        srcs, dsts, rowss = (sv0, sv1), (dv0, dv1, dv2), (rw0, rw1)
        isrcs, idsts = (isr0, isr1), (idt0, idt1, idt2)
        gsems, ssems = (gs0, gs1), (ss0, ss1, ss2)
        c, s, w = _worker_ids()
        r0 = s * R_MAIN
        e0 = w * EPW
        pltpu.sync_copy(z128_hbm, rw0)
        _zero_shared(accs, rw0, r0)

        @pl.when(s == 0)
        def _():
            pltpu.sync_copy(rw0.at[pl.ds(0, R_TAIL)],
                            accs.at[pl.ds(R_TAIL_BASE, R_TAIL)])

        plsc.subcore_barrier()

        def src_start(j, b2):
            pltpu.async_copy(src_hbm.at[pl.ds(e0 + j * CHUNK, CHUNK)],
                             srcs[b2], isrcs[b2])

        def src_wait(b2):
            pltpu.make_async_copy(src_hbm.at[pl.ds(0, CHUNK)], srcs[b2],
                                  isrcs[b2]).wait()

        def dst_start(j, b3):
            pltpu.async_copy(dst_hbm.at[pl.ds(e0 + j * CHUNK, CHUNK)],
                             dsts[b3], idsts[b3])

        def dst_wait(b3):
            pltpu.make_async_copy(dst_hbm.at[pl.ds(0, CHUNK)], dsts[b3],
                                  idsts[b3]).wait()

        def gather_start(b2):
            pltpu.async_copy(x_hbm.at[srcs[b2]], rowss[b2], gsems[b2])

        def gather_wait(b2):
            pltpu.make_async_copy(x_hbm.at[srcs[b2]], rowss[b2],
                                  gsems[b2]).wait()

        def scatter_start(b2, b3):
            pltpu.async_copy(rowss[b2], accs.at[dsts[b3]], ssems[b3],
                             add=True)

        def scatter_wait(b3):
            pltpu.make_async_copy(rw0, accs.at[dsts[b3]], ssems[b3]).wait()

        def turn(j, b2, b3, has_prev, start_next, prefetch_j):
            gather_wait(b2)
            scatter_start(b2, b3)
            if has_prev:
                scatter_wait((b3 + 2) % 3)
            if start_next:
                src_wait(1 - b2)
                dst_wait((b3 + 1) % 3)
                gather_start(1 - b2)
            if prefetch_j is not None:
                src_start(prefetch_j, b2)
                dst_start(prefetch_j, (b3 + 2) % 3)

        src_start(0, 0)
        dst_start(0, 0)
        src_start(1, 1)
        dst_start(1, 1)
        src_wait(0)
        dst_wait(0)
        gather_start(0)
        turn(0, 0, 0, False, True, 2)
        turn(1, 1, 1, True, True, 3)
        turn(2, 0, 2, True, True, 4)
        turn(3, 1, 0, True, True, 5)

        def loop_body(i, carry):
            j0 = 4 + i * 6
            turn(j0, 0, 1, True, True, j0 + 2)
            turn(j0 + 1, 1, 2, True, True, j0 + 3)
            turn(j0 + 2, 0, 0, True, True, j0 + 4)
            turn(j0 + 3, 1, 1, True, True, j0 + 5)
            turn(j0 + 4, 0, 2, True, True, j0 + 6)
            turn(j0 + 5, 1, 0, True, True, j0 + 7)
            return carry

        lax.fori_loop(0, (FULL - 6) // 6, loop_body, 0)
        turn(FULL - 2, 0, (FULL - 2) % 3, True, True, None)
        turn(FULL - 1, 1, (FULL - 1) % 3, True, False, None)
        scatter_wait((FULL - 1) % 3)
        pltpu.sync_copy(src_hbm.at[pl.ds(e0 + FULL * CHUNK, TAIL)], srct)
        pltpu.sync_copy(dst_hbm.at[pl.ds(e0 + FULL * CHUNK, TAIL)], dstt)
        pltpu.async_copy(x_hbm.at[srct], rw0.at[pl.ds(0, TAIL)], gs0).wait()
        pltpu.sync_copy(rw0.at[pl.ds(0, TAIL)], accs.at[dstt], add=True)

        plsc.subcore_barrier()
        one = _flush_shared(accs, rw0, acc_out, c, r0)

        @pl.when(s == 0)
        def _():
            one(R_TAIL_BASE, R_TAIL)

    return seg_sum(x, src, dst, z128)


def _sc_degree(dst, z16, ones16):
    mesh = plsc.VectorSubcoreMesh(core_axis_name="c", subcore_axis_name="s")

    @functools.partial(
        pl.kernel,
        out_type=jax.ShapeDtypeStruct((NC, N_NODES, DEG_W), jnp.float32),
        mesh=mesh,
        scratch_types=[
            pltpu.VMEM((CHUNK,), jnp.int32),
            pltpu.VMEM((CHUNK,), jnp.int32),
            pltpu.VMEM((CHUNK,), jnp.int32),
            pltpu.VMEM((TAIL,), jnp.int32),
            pltpu.VMEM((CHUNK, DEG_W), jnp.float32),
            pltpu.VMEM((CHUNK, DEG_W), jnp.float32),
            pltpu.VMEM_SHARED((N_NODES, DEG_W), jnp.float32),
            pltpu.SemaphoreType.DMA,
            pltpu.SemaphoreType.DMA,
            pltpu.SemaphoreType.DMA,
            pltpu.SemaphoreType.DMA,
            pltpu.SemaphoreType.DMA,
            pltpu.SemaphoreType.DMA,
        ],
    )
    def deg_sum(dst_hbm, z16_hbm, ones_hbm, deg_out,
                dv0, dv1, dv2, dstt, zb16, onesv, degs,
                is0, is1, is2, ss0, ss1, ss2):
        dsts = (dv0, dv1, dv2)
        isems, ssems = (is0, is1, is2), (ss0, ss1, ss2)
        c, s, w = _worker_ids()
        r0 = s * R_MAIN
        e0 = w * EPW
        pltpu.sync_copy(z16_hbm, zb16)
        pltpu.sync_copy(ones_hbm, onesv)
        _zero_shared(degs, zb16, r0)

        @pl.when(s == 0)
        def _():
            pltpu.sync_copy(zb16.at[pl.ds(0, R_TAIL)],
                            degs.at[pl.ds(R_TAIL_BASE, R_TAIL)])

        plsc.subcore_barrier()

        def idx_start(j, b):
            pltpu.async_copy(dst_hbm.at[pl.ds(e0 + j * CHUNK, CHUNK)],
                             dsts[b], isems[b])

        def idx_wait(b):
            pltpu.make_async_copy(dst_hbm.at[pl.ds(0, CHUNK)], dsts[b],
                                  isems[b]).wait()

        def scatter_start(b):
            pltpu.async_copy(onesv, degs.at[dsts[b]], ssems[b], add=True)

        def scatter_wait(b):
            pltpu.make_async_copy(onesv, degs.at[dsts[b]], ssems[b]).wait()

        def turn(j, b, prefetch_j):
            idx_wait(b)
            scatter_start(b)
            bp = (b + 1) % 3
            if prefetch_j is not None:
                scatter_wait(bp)
                idx_start(prefetch_j, bp)

        idx_start(0, 0)
        idx_start(1, 1)
        idx_start(2, 2)
        turn(0, 0, None)
        turn(1, 1, None)

        def loop_body(i, carry):
            j0 = i * 3
            turn(j0 + 2, 2, j0 + 3)
            turn(j0 + 3, 0, j0 + 4)
            turn(j0 + 4, 1, j0 + 5)
            return carry

        lax.fori_loop(0, (FULL - 2) // 3, loop_body, 0)
        turn(FULL - 1, 2, None)
        scatter_wait(0)
        scatter_wait(1)
        scatter_wait(2)
        pltpu.sync_copy(dst_hbm.at[pl.ds(e0 + FULL * CHUNK, TAIL)], dstt)
        pltpu.sync_copy(onesv.at[pl.ds(0, TAIL)], degs.at[dstt], add=True)
        plsc.subcore_barrier()
        one = _flush_shared(degs, zb16, deg_out, c, r0)

        @pl.when(s == 0)
        def _():
            one(R_TAIL_BASE, R_TAIL)

    return deg_sum(dst, z16, ones16)


def _tc_body(p_ref, d_ref, w_ref, b_ref, g_ref, be_ref, o_ref):
    agg = p_ref[0] + p_ref[1]
    deg = d_ref[0, :, 0:1] + d_ref[1, :, 0:1]
    lin = lax.dot_general(
        agg, w_ref[...], (((1,), (1,)), ((), ())),
        precision=lax.Precision.HIGHEST,
        preferred_element_type=jnp.float32,
    )
    lin = lin + deg * b_ref[...]
    mean = jnp.mean(lin, axis=1, keepdims=True)
    cent = lin - mean
    var = jnp.mean(cent * cent, axis=1, keepdims=True)
    normed = cent * lax.rsqrt(var + EPS)
    o_ref[...] = normed * g_ref[...] + be_ref[...]


def _tc_finish(partials, degp, W, b, gamma, beta):
    B = 1000
    grid = N_NODES // B
    return pl.pallas_call(
        _tc_body,
        grid=(grid,),
        in_specs=[
            pl.BlockSpec((NC, B, D), lambda i: (0, i, 0)),
            pl.BlockSpec((NC, B, DEG_W), lambda i: (0, i, 0)),
            pl.BlockSpec((D, D), lambda i: (0, 0)),
            pl.BlockSpec((1, D), lambda i: (0, 0)),
            pl.BlockSpec((1, D), lambda i: (0, 0)),
            pl.BlockSpec((1, D), lambda i: (0, 0)),
        ],
        out_specs=pl.BlockSpec((B, D), lambda i: (i, 0)),
        out_shape=jax.ShapeDtypeStruct((N_NODES, D), jnp.float32),
    )(partials, degp, W, b, gamma, beta)


def kernel(x, edge_index, W, b, gamma, beta):
    src = edge_index[0]
    dst = edge_index[1]
    z128 = jnp.zeros((CHUNK, D), jnp.float32)
    z16 = jnp.zeros((CHUNK, DEG_W), jnp.float32)
    ones16 = jnp.ones((CHUNK, DEG_W), jnp.float32)
    partials = _sc_segment_sum(x, src, dst, z128)
    degp = _sc_degree(dst, z16, ones16)
    return _tc_finish(partials, degp, W,
                      b.reshape(1, D), gamma.reshape(1, D), beta.reshape(1, D))


# speedup vs baseline: 1.0265x; 1.0265x over previous
import functools

import jax
import jax.numpy as jnp
from jax import lax
from jax.experimental import pallas as pl
from jax.experimental.pallas import tpu as pltpu
from jax.experimental.pallas import tpu_sc as plsc

N_NODES = 10000
N_EDGES = 320000
D = 128
DEG_W = 128
EPS = 1e-5

NC = 2
NS = 16
NW = NC * NS
CHUNK = 128
N_CHUNKS = N_EDGES // CHUNK
BASE_CHUNKS = N_CHUNKS // NW
EXTRA = N_CHUNKS - BASE_CHUNKS * NW
EPW = N_EDGES // NW
FULL = EPW // CHUNK
TAIL = EPW - FULL * CHUNK
R_MAIN = 624
R_TAIL_BASE = NS * R_MAIN
R_TAIL = N_NODES - R_TAIL_BASE


def _worker_ids():
    c = lax.axis_index("c")
    s = lax.axis_index("s")
    return c, s, s * NC + c


def _zero_shared(shared, stage, r0):
    for k in range(4):
        pltpu.sync_copy(stage, shared.at[pl.ds(r0 + k * CHUNK, CHUNK)])
    pltpu.sync_copy(stage.at[pl.ds(0, R_MAIN - 4 * CHUNK)],
                    shared.at[pl.ds(r0 + 4 * CHUNK, R_MAIN - 4 * CHUNK)])


def _flush_shared(shared, stage, out_hbm, c, r0):
    def one(base, n):
        pltpu.sync_copy(shared.at[pl.ds(base, n)], stage.at[pl.ds(0, n)])
        pltpu.sync_copy(stage.at[pl.ds(0, n)], out_hbm.at[c, pl.ds(base, n)])

    for k in range(4):
        one(r0 + k * CHUNK, CHUNK)
    one(r0 + 4 * CHUNK, R_MAIN - 4 * CHUNK)
    return one


def _sc_segment_sum(x, src, dst, z128):
    mesh = plsc.VectorSubcoreMesh(core_axis_name="c", subcore_axis_name="s")

    @functools.partial(
        pl.kernel,
        out_type=jax.ShapeDtypeStruct((NC, N_NODES, D), jnp.float32),
        mesh=mesh,
        scratch_types=[
            pltpu.VMEM((CHUNK,), jnp.int32),
            pltpu.VMEM((CHUNK,), jnp.int32),
            pltpu.VMEM((CHUNK,), jnp.int32),
            pltpu.VMEM((CHUNK,), jnp.int32),
            pltpu.VMEM((CHUNK, D), jnp.float32),
            pltpu.VMEM((CHUNK, D), jnp.float32),
            pltpu.VMEM((TAIL,), jnp.int32),
            pltpu.VMEM((TAIL,), jnp.int32),
            pltpu.VMEM_SHARED((N_NODES, D), jnp.float32),
            pltpu.SemaphoreType.DMA,
            pltpu.SemaphoreType.DMA,
            pltpu.SemaphoreType.DMA,
            pltpu.SemaphoreType.DMA,
        ],
    )
    def seg_sum(x_hbm, src_hbm, dst_hbm, z128_hbm, acc_out,
                sv0, sv1, dv0, dv1, rw0, rw1, srct, dstt, accs,
                is0, is1, gs0, gs1):
        srcs, dsts, rowss = (sv0, sv1), (dv0, dv1), (rw0, rw1)
        isems, gsems = (is0, is1), (gs0, gs1)
        c, s, w = _worker_ids()
        r0 = s * R_MAIN
        e0 = w * EPW
        pltpu.sync_copy(z128_hbm, rw0)
        _zero_shared(accs, rw0, r0)

        @pl.when(s == 0)
        def _():
            pltpu.sync_copy(rw0.at[pl.ds(0, R_TAIL)],
                            accs.at[pl.ds(R_TAIL_BASE, R_TAIL)])

        plsc.subcore_barrier()

        def idx_start(j, b):
            pltpu.async_copy(src_hbm.at[pl.ds(e0 + j * CHUNK, CHUNK)],
                             srcs[b], isems[b])
            pltpu.async_copy(dst_hbm.at[pl.ds(e0 + j * CHUNK, CHUNK)],
                             dsts[b], isems[b])

        def idx_wait(b):
            pltpu.make_async_copy(src_hbm.at[pl.ds(0, CHUNK)], srcs[b],
                                  isems[b]).wait()
            pltpu.make_async_copy(dst_hbm.at[pl.ds(0, CHUNK)], dsts[b],
                                  isems[b]).wait()

        def gather_start(b):
            pltpu.async_copy(x_hbm.at[srcs[b]], rowss[b], gsems[b])

        def gather_wait(b):
            pltpu.make_async_copy(x_hbm.at[srcs[b]], rowss[b],
                                  gsems[b]).wait()

        def scatter(b):
            pltpu.sync_copy(rowss[b], accs.at[dsts[b]], add=True)

        def turn(j, b, start_next, prefetch_j):
            if start_next:
                idx_wait(1 - b)
                gather_start(1 - b)
            gather_wait(b)
            scatter(b)
            if prefetch_j is not None:
                idx_start(prefetch_j, b)

        idx_start(0, 0)
        idx_start(1, 1)
        idx_wait(0)
        gather_start(0)

        def loop_body(i, carry):
            j0 = i * 2
            turn(j0, 0, True, j0 + 2)
            turn(j0 + 1, 1, True, j0 + 3)
            return carry

        lax.fori_loop(0, FULL // 2 - 1, loop_body, 0)
        turn(FULL - 2, 0, True, None)
        turn(FULL - 1, 1, False, None)
        pltpu.sync_copy(src_hbm.at[pl.ds(e0 + FULL * CHUNK, TAIL)], srct)
        pltpu.sync_copy(dst_hbm.at[pl.ds(e0 + FULL * CHUNK, TAIL)], dstt)
        pltpu.async_copy(x_hbm.at[srct], rw0.at[pl.ds(0, TAIL)], gs0).wait()
        pltpu.sync_copy(rw0.at[pl.ds(0, TAIL)], accs.at[dstt], add=True)

        plsc.subcore_barrier()
        one = _flush_shared(accs, rw0, acc_out, c, r0)

        @pl.when(s == 0)
        def _():
            one(R_TAIL_BASE, R_TAIL)

    return seg_sum(x, src, dst, z128)


def _sc_degree(dst, z16, ones16):
    mesh = plsc.VectorSubcoreMesh(core_axis_name="c", subcore_axis_name="s")

    @functools.partial(
        pl.kernel,
        out_type=jax.ShapeDtypeStruct((NC, N_NODES, DEG_W), jnp.float32),
        mesh=mesh,
        scratch_types=[
            pltpu.VMEM((CHUNK,), jnp.int32),
            pltpu.VMEM((CHUNK,), jnp.int32),
            pltpu.VMEM((CHUNK,), jnp.int32),
            pltpu.VMEM((TAIL,), jnp.int32),
            pltpu.VMEM((CHUNK, DEG_W), jnp.float32),
            pltpu.VMEM((CHUNK, DEG_W), jnp.float32),
            pltpu.VMEM_SHARED((N_NODES, DEG_W), jnp.float32),
            pltpu.SemaphoreType.DMA,
            pltpu.SemaphoreType.DMA,
            pltpu.SemaphoreType.DMA,
            pltpu.SemaphoreType.DMA,
            pltpu.SemaphoreType.DMA,
            pltpu.SemaphoreType.DMA,
        ],
    )
    def deg_sum(dst_hbm, z16_hbm, ones_hbm, deg_out,
                dv0, dv1, dv2, dstt, zb16, onesv, degs,
                is0, is1, is2, ss0, ss1, ss2):
        dsts = (dv0, dv1, dv2)
        isems, ssems = (is0, is1, is2), (ss0, ss1, ss2)
        c, s, w = _worker_ids()
        r0 = s * R_MAIN
        e0 = w * EPW
        pltpu.sync_copy(z16_hbm, zb16)
        pltpu.sync_copy(ones_hbm, onesv)
        _zero_shared(degs, zb16, r0)

        @pl.when(s == 0)
        def _():
            pltpu.sync_copy(zb16.at[pl.ds(0, R_TAIL)],
                            degs.at[pl.ds(R_TAIL_BASE, R_TAIL)])

        plsc.subcore_barrier()

        def idx_start(j, b):
            pltpu.async_copy(dst_hbm.at[pl.ds(e0 + j * CHUNK, CHUNK)],
                             dsts[b], isems[b])

        def idx_wait(b):
            pltpu.make_async_copy(dst_hbm.at[pl.ds(0, CHUNK)], dsts[b],
                                  isems[b]).wait()

        def scatter_start(b):
            pltpu.async_copy(onesv, degs.at[dsts[b]], ssems[b], add=True)

        def scatter_wait(b):
            pltpu.make_async_copy(onesv, degs.at[dsts[b]], ssems[b]).wait()

        def turn(j, b, prefetch_j):
            idx_wait(b)
            scatter_start(b)
            bp = (b + 1) % 3
            if prefetch_j is not None:
                scatter_wait(bp)
                idx_start(prefetch_j, bp)

        idx_start(0, 0)
        idx_start(1, 1)
        idx_start(2, 2)
        turn(0, 0, None)
        turn(1, 1, None)

        def loop_body(i, carry):
            j0 = i * 3
            turn(j0 + 2, 2, j0 + 3)
            turn(j0 + 3, 0, j0 + 4)
            turn(j0 + 4, 1, j0 + 5)
            return carry

        lax.fori_loop(0, (FULL - 2) // 3, loop_body, 0)
        turn(FULL - 1, 2, None)
        scatter_wait(0)
        scatter_wait(1)
        scatter_wait(2)
        pltpu.sync_copy(dst_hbm.at[pl.ds(e0 + FULL * CHUNK, TAIL)], dstt)
        pltpu.sync_copy(onesv.at[pl.ds(0, TAIL)], degs.at[dstt], add=True)
        plsc.subcore_barrier()
        one = _flush_shared(degs, zb16, deg_out, c, r0)

        @pl.when(s == 0)
        def _():
            one(R_TAIL_BASE, R_TAIL)

    return deg_sum(dst, z16, ones16)


def _tc_body(p_ref, d_ref, w_ref, b_ref, g_ref, be_ref, o_ref):
    agg = p_ref[0] + p_ref[1]
    deg = d_ref[0, :, 0:1] + d_ref[1, :, 0:1]
    lin = lax.dot_general(
        agg, w_ref[...], (((1,), (1,)), ((), ())),
        precision=lax.Precision.HIGHEST,
        preferred_element_type=jnp.float32,
    )
    lin = lin + deg * b_ref[...]
    mean = jnp.mean(lin, axis=1, keepdims=True)
    cent = lin - mean
    var = jnp.mean(cent * cent, axis=1, keepdims=True)
    normed = cent * lax.rsqrt(var + EPS)
    o_ref[...] = normed * g_ref[...] + be_ref[...]


def _tc_finish(partials, degp, W, b, gamma, beta):
    B = 1000
    grid = N_NODES // B
    return pl.pallas_call(
        _tc_body,
        grid=(grid,),
        in_specs=[
            pl.BlockSpec((NC, B, D), lambda i: (0, i, 0)),
            pl.BlockSpec((NC, B, DEG_W), lambda i: (0, i, 0)),
            pl.BlockSpec((D, D), lambda i: (0, 0)),
            pl.BlockSpec((1, D), lambda i: (0, 0)),
            pl.BlockSpec((1, D), lambda i: (0, 0)),
            pl.BlockSpec((1, D), lambda i: (0, 0)),
        ],
        out_specs=pl.BlockSpec((B, D), lambda i: (i, 0)),
        out_shape=jax.ShapeDtypeStruct((N_NODES, D), jnp.float32),
    )(partials, degp, W, b, gamma, beta)


def kernel(x, edge_index, W, b, gamma, beta):
    src = edge_index[0]
    dst = edge_index[1]
    z128 = jnp.zeros((CHUNK, D), jnp.float32)
    z16 = jnp.zeros((CHUNK, DEG_W), jnp.float32)
    ones16 = jnp.ones((CHUNK, DEG_W), jnp.float32)
    partials = _sc_segment_sum(x, src, dst, z128)
    degp = _sc_degree(dst, z16, ones16)
    return _tc_finish(partials, degp, W,
                      b.reshape(1, D), gamma.reshape(1, D), beta.reshape(1, D))


# speedup vs baseline: 1.0984x; 1.0700x over previous
import functools

import jax
import jax.numpy as jnp
from jax import lax
from jax.experimental import pallas as pl
from jax.experimental.pallas import tpu as pltpu
from jax.experimental.pallas import tpu_sc as plsc

N_NODES = 10000
N_EDGES = 320000
D = 128
DEG_W = 64
EPS = 1e-5

NC = 2
NS = 16
NW = NC * NS
CHUNK = 128
N_CHUNKS = N_EDGES // CHUNK
BASE_CHUNKS = N_CHUNKS // NW
EXTRA = N_CHUNKS - BASE_CHUNKS * NW
EPW = N_EDGES // NW
FULL = EPW // CHUNK
TAIL = EPW - FULL * CHUNK
R_MAIN = 624
R_TAIL_BASE = NS * R_MAIN
R_TAIL = N_NODES - R_TAIL_BASE


def _worker_ids():
    c = lax.axis_index("c")
    s = lax.axis_index("s")
    return c, s, s * NC + c


def _zero_shared(shared, stage, r0):
    for k in range(4):
        pltpu.sync_copy(stage, shared.at[pl.ds(r0 + k * CHUNK, CHUNK)])
    pltpu.sync_copy(stage.at[pl.ds(0, R_MAIN - 4 * CHUNK)],
                    shared.at[pl.ds(r0 + 4 * CHUNK, R_MAIN - 4 * CHUNK)])


def _flush_shared(shared, stage, out_hbm, c, r0):
    def one(base, n):
        pltpu.sync_copy(shared.at[pl.ds(base, n)], stage.at[pl.ds(0, n)])
        pltpu.sync_copy(stage.at[pl.ds(0, n)], out_hbm.at[c, pl.ds(base, n)])

    for k in range(4):
        one(r0 + k * CHUNK, CHUNK)
    one(r0 + 4 * CHUNK, R_MAIN - 4 * CHUNK)
    return one


def _sc_segment_sum(x, src, dst, z128):
    mesh = plsc.VectorSubcoreMesh(core_axis_name="c", subcore_axis_name="s")

    @functools.partial(
        pl.kernel,
        out_type=jax.ShapeDtypeStruct((NC, N_NODES, D), jnp.float32),
        mesh=mesh,
        scratch_types=[
            pltpu.VMEM((CHUNK,), jnp.int32),
            pltpu.VMEM((CHUNK,), jnp.int32),
            pltpu.VMEM((CHUNK,), jnp.int32),
            pltpu.VMEM((CHUNK,), jnp.int32),
            pltpu.VMEM((CHUNK, D), jnp.float32),
            pltpu.VMEM((CHUNK, D), jnp.float32),
            pltpu.VMEM((TAIL,), jnp.int32),
            pltpu.VMEM((TAIL,), jnp.int32),
            pltpu.VMEM_SHARED((N_NODES, D), jnp.float32),
            pltpu.SemaphoreType.DMA,
            pltpu.SemaphoreType.DMA,
            pltpu.SemaphoreType.DMA,
            pltpu.SemaphoreType.DMA,
        ],
    )
    def seg_sum(x_hbm, src_hbm, dst_hbm, z128_hbm, acc_out,
                sv0, sv1, dv0, dv1, rw0, rw1, srct, dstt, accs,
                is0, is1, gs0, gs1):
        srcs, dsts, rowss = (sv0, sv1), (dv0, dv1), (rw0, rw1)
        isems, gsems = (is0, is1), (gs0, gs1)
        c, s, w = _worker_ids()
        r0 = s * R_MAIN
        e0 = w * EPW
        pltpu.sync_copy(z128_hbm, rw0)
        _zero_shared(accs, rw0, r0)

        @pl.when(s == 0)
        def _():
            pltpu.sync_copy(rw0.at[pl.ds(0, R_TAIL)],
                            accs.at[pl.ds(R_TAIL_BASE, R_TAIL)])

        plsc.subcore_barrier()

        def idx_start(j, b):
            pltpu.async_copy(src_hbm.at[pl.ds(e0 + j * CHUNK, CHUNK)],
                             srcs[b], isems[b])
            pltpu.async_copy(dst_hbm.at[pl.ds(e0 + j * CHUNK, CHUNK)],
                             dsts[b], isems[b])

        def idx_wait(b):
            pltpu.make_async_copy(src_hbm.at[pl.ds(0, CHUNK)], srcs[b],
                                  isems[b]).wait()
            pltpu.make_async_copy(dst_hbm.at[pl.ds(0, CHUNK)], dsts[b],
                                  isems[b]).wait()

        def gather_start(b):
            pltpu.async_copy(x_hbm.at[srcs[b]], rowss[b], gsems[b])

        def gather_wait(b):
            pltpu.make_async_copy(x_hbm.at[srcs[b]], rowss[b],
                                  gsems[b]).wait()

        def scatter(b):
            pltpu.sync_copy(rowss[b], accs.at[dsts[b]], add=True)

        def turn(j, b, start_next, prefetch_j):
            if start_next:
                idx_wait(1 - b)
                gather_start(1 - b)
            gather_wait(b)
            scatter(b)
            if prefetch_j is not None:
                idx_start(prefetch_j, b)

        idx_start(0, 0)
        idx_start(1, 1)
        idx_wait(0)
        gather_start(0)

        def loop_body(i, carry):
            j0 = i * 2
            turn(j0, 0, True, j0 + 2)
            turn(j0 + 1, 1, True, j0 + 3)
            return carry

        lax.fori_loop(0, FULL // 2 - 1, loop_body, 0)
        turn(FULL - 2, 0, True, None)
        turn(FULL - 1, 1, False, None)
        pltpu.sync_copy(src_hbm.at[pl.ds(e0 + FULL * CHUNK, TAIL)], srct)
        pltpu.sync_copy(dst_hbm.at[pl.ds(e0 + FULL * CHUNK, TAIL)], dstt)
        pltpu.async_copy(x_hbm.at[srct], rw0.at[pl.ds(0, TAIL)], gs0).wait()
        pltpu.sync_copy(rw0.at[pl.ds(0, TAIL)], accs.at[dstt], add=True)

        plsc.subcore_barrier()
        one = _flush_shared(accs, rw0, acc_out, c, r0)

        @pl.when(s == 0)
        def _():
            one(R_TAIL_BASE, R_TAIL)

    return seg_sum(x, src, dst, z128)


def _sc_degree(dst, z16, ones16):
    mesh = plsc.VectorSubcoreMesh(core_axis_name="c", subcore_axis_name="s")

    @functools.partial(
        pl.kernel,
        out_type=jax.ShapeDtypeStruct((NC, N_NODES, DEG_W), jnp.float32),
        mesh=mesh,
        scratch_types=[
            pltpu.VMEM((CHUNK,), jnp.int32),
            pltpu.VMEM((CHUNK,), jnp.int32),
            pltpu.VMEM((CHUNK,), jnp.int32),
            pltpu.VMEM((TAIL,), jnp.int32),
            pltpu.VMEM((CHUNK, DEG_W), jnp.float32),
            pltpu.VMEM((CHUNK, DEG_W), jnp.float32),
            pltpu.VMEM_SHARED((N_NODES, DEG_W), jnp.float32),
            pltpu.SemaphoreType.DMA,
            pltpu.SemaphoreType.DMA,
            pltpu.SemaphoreType.DMA,
            pltpu.SemaphoreType.DMA,
            pltpu.SemaphoreType.DMA,
            pltpu.SemaphoreType.DMA,
        ],
    )
    def deg_sum(dst_hbm, z16_hbm, ones_hbm, deg_out,
                dv0, dv1, dv2, dstt, zb16, onesv, degs,
                is0, is1, is2, ss0, ss1, ss2):
        dsts = (dv0, dv1, dv2)
        isems, ssems = (is0, is1, is2), (ss0, ss1, ss2)
        c, s, w = _worker_ids()
        r0 = s * R_MAIN
        e0 = w * EPW
        pltpu.sync_copy(z16_hbm, zb16)
        pltpu.sync_copy(ones_hbm, onesv)
        _zero_shared(degs, zb16, r0)

        @pl.when(s == 0)
        def _():
            pltpu.sync_copy(zb16.at[pl.ds(0, R_TAIL)],
                            degs.at[pl.ds(R_TAIL_BASE, R_TAIL)])

        plsc.subcore_barrier()

        def idx_start(j, b):
            pltpu.async_copy(dst_hbm.at[pl.ds(e0 + j * CHUNK, CHUNK)],
                             dsts[b], isems[b])

        def idx_wait(b):
            pltpu.make_async_copy(dst_hbm.at[pl.ds(0, CHUNK)], dsts[b],
                                  isems[b]).wait()

        def scatter_start(b):
            pltpu.async_copy(onesv, degs.at[dsts[b]], ssems[b], add=True)

        def scatter_wait(b):
            pltpu.make_async_copy(onesv, degs.at[dsts[b]], ssems[b]).wait()

        def turn(j, b, prefetch_j):
            idx_wait(b)
            scatter_start(b)
            bp = (b + 1) % 3
            if prefetch_j is not None:
                scatter_wait(bp)
                idx_start(prefetch_j, bp)

        idx_start(0, 0)
        idx_start(1, 1)
        idx_start(2, 2)
        turn(0, 0, None)
        turn(1, 1, None)

        def loop_body(i, carry):
            j0 = i * 3
            turn(j0 + 2, 2, j0 + 3)
            turn(j0 + 3, 0, j0 + 4)
            turn(j0 + 4, 1, j0 + 5)
            return carry

        lax.fori_loop(0, (FULL - 2) // 3, loop_body, 0)
        turn(FULL - 1, 2, None)
        scatter_wait(0)
        scatter_wait(1)
        scatter_wait(2)
        pltpu.sync_copy(dst_hbm.at[pl.ds(e0 + FULL * CHUNK, TAIL)], dstt)
        pltpu.sync_copy(onesv.at[pl.ds(0, TAIL)], degs.at[dstt], add=True)
        plsc.subcore_barrier()
        one = _flush_shared(degs, zb16, deg_out, c, r0)

        @pl.when(s == 0)
        def _():
            one(R_TAIL_BASE, R_TAIL)

    return deg_sum(dst, z16, ones16)


def _tc_body(p_ref, d_ref, w_ref, b_ref, g_ref, be_ref, o_ref):
    agg = p_ref[0] + p_ref[1]
    deg = d_ref[0] + d_ref[1]
    lin = lax.dot_general(
        agg, w_ref[...], (((1,), (1,)), ((), ())),
        precision=lax.Precision.HIGHEST,
        preferred_element_type=jnp.float32,
    )
    lin = lin + deg * b_ref[...]
    mean = jnp.mean(lin, axis=1, keepdims=True)
    cent = lin - mean
    var = jnp.mean(cent * cent, axis=1, keepdims=True)
    normed = cent * lax.rsqrt(var + EPS)
    o_ref[...] = normed * g_ref[...] + be_ref[...]


def _tc_finish(partials, degp, W, b, gamma, beta):
    B = 1000
    grid = N_NODES // B
    return pl.pallas_call(
        _tc_body,
        grid=(grid,),
        in_specs=[
            pl.BlockSpec((NC, B, D), lambda i: (0, i, 0)),
            pl.BlockSpec((NC, B, 1), lambda i: (0, i, 0)),
            pl.BlockSpec((D, D), lambda i: (0, 0)),
            pl.BlockSpec((1, D), lambda i: (0, 0)),
            pl.BlockSpec((1, D), lambda i: (0, 0)),
            pl.BlockSpec((1, D), lambda i: (0, 0)),
        ],
        out_specs=pl.BlockSpec((B, D), lambda i: (i, 0)),
        out_shape=jax.ShapeDtypeStruct((N_NODES, D), jnp.float32),
    )(partials, degp, W, b, gamma, beta)


def kernel(x, edge_index, W, b, gamma, beta):
    src = edge_index[0]
    dst = edge_index[1]
    z128 = jnp.zeros((CHUNK, D), jnp.float32)
    z16 = jnp.zeros((CHUNK, DEG_W), jnp.float32)
    ones16 = jnp.ones((CHUNK, DEG_W), jnp.float32)
    partials = _sc_segment_sum(x, src, dst, z128)
    degp = _sc_degree(dst, z16, ones16)[:, :, :1]
    return _tc_finish(partials, degp, W,
                      b.reshape(1, D), gamma.reshape(1, D), beta.reshape(1, D))
